# Initial kernel scaffold; baseline (speedup 1.0000x reference)
import jax
import jax.numpy as jnp
from jax.experimental import pallas as pl


def kernel(x, table):
    raise NotImplementedError("write your pallas kernel here")


# speedup vs baseline: 1.6568x; 1.6568x over previous
import functools

import jax
import jax.numpy as jnp
from jax import lax
from jax.experimental import pallas as pl
from jax.experimental.pallas import tpu as pltpu
from jax.experimental.pallas import tpu_sc as plsc

_NUM_CORES = 2
_NUM_SUBCORES = 16
_NUM_WORKERS = _NUM_CORES * _NUM_SUBCORES


def _emb_lookup(total, d, *, chunk):
    b_per_w = total // _NUM_WORKERS
    n_chunks = b_per_w // chunk

    mesh = plsc.VectorSubcoreMesh(core_axis_name="c", subcore_axis_name="s")

    @functools.partial(
        pl.kernel,
        mesh=mesh,
        out_type=jax.ShapeDtypeStruct((total, d), jnp.float32),
        scratch_types=[
            pltpu.VMEM((n_chunks, chunk), jnp.int32),
            pltpu.VMEM((chunk, d), jnp.float32),
            pltpu.VMEM((chunk, d), jnp.float32),
            pltpu.SemaphoreType.DMA,
            pltpu.SemaphoreType.DMA,
        ],
    )
    def emb(x_hbm, table_hbm, out_hbm, idx_v, rows0, rows1, gsem, osem):
        wid = lax.axis_index("s") * _NUM_CORES + lax.axis_index("c")
        base = wid * b_per_w
        pltpu.sync_copy(x_hbm.at[wid], idx_v)

        bufs = (rows0, rows1)
        gather = pltpu.async_copy(table_hbm.at[idx_v.at[0]], bufs[0], gsem)
        out_cp = None
        for j in range(n_chunks):
            gather.wait()
            if out_cp is not None:
                out_cp.wait()
            if j + 1 < n_chunks:
                gather = pltpu.async_copy(
                    table_hbm.at[idx_v.at[j + 1]], bufs[(j + 1) % 2], gsem
                )
            out_cp = pltpu.async_copy(
                bufs[j % 2], out_hbm.at[pl.ds(base + j * chunk, chunk)], osem
            )
        out_cp.wait()

    return emb


def kernel(x, table):
    b, s = x.shape
    total = b * s
    d = table.shape[1]
    chunk = 64
    b_per_w = total // _NUM_WORKERS
    n_chunks = b_per_w // chunk
    x_grid = x.reshape(_NUM_WORKERS, n_chunks, chunk).astype(jnp.int32)
    out = _emb_lookup(total, d, chunk=chunk)(x_grid, table)
    return out.reshape(b, s, d)


# speedup vs baseline: 1.6691x; 1.0074x over previous
import functools

import jax
import jax.numpy as jnp
from jax import lax
from jax.experimental import pallas as pl
from jax.experimental.pallas import tpu as pltpu
from jax.experimental.pallas import tpu_sc as plsc

_NUM_CORES = 2
_NUM_SUBCORES = 16
_NUM_WORKERS = _NUM_CORES * _NUM_SUBCORES


def _emb_lookup(total, d, *, chunk):
    b_per_w = total // _NUM_WORKERS
    n_chunks = b_per_w // chunk

    mesh = plsc.VectorSubcoreMesh(core_axis_name="c", subcore_axis_name="s")

    nbuf = 4

    @functools.partial(
        pl.kernel,
        mesh=mesh,
        out_type=jax.ShapeDtypeStruct((total, d), jnp.float32),
        scratch_types=[
            pltpu.VMEM((n_chunks, chunk), jnp.int32),
            [pltpu.VMEM((chunk, d), jnp.float32) for _ in range(nbuf)],
            pltpu.SemaphoreType.DMA,
            pltpu.SemaphoreType.DMA,
        ],
    )
    def emb(x_hbm, table_hbm, out_hbm, idx_v, bufs, gsem, osem):
        wid = lax.axis_index("s") * _NUM_CORES + lax.axis_index("c")
        base = wid * b_per_w
        pltpu.sync_copy(x_hbm.at[wid], idx_v)

        gathers = [None] * n_chunks
        writes = [None] * n_chunks
        for j in range(nbuf - 1):
            gathers[j] = pltpu.async_copy(
                table_hbm.at[idx_v.at[j]], bufs[j % nbuf], gsem
            )
        for j in range(n_chunks):
            gathers[j].wait()
            nxt = j + nbuf - 1
            if nxt < n_chunks:
                if writes[nxt - nbuf] is not None:
                    writes[nxt - nbuf].wait()
                gathers[nxt] = pltpu.async_copy(
                    table_hbm.at[idx_v.at[nxt]], bufs[nxt % nbuf], gsem
                )
            writes[j] = pltpu.async_copy(
                bufs[j % nbuf], out_hbm.at[pl.ds(base + j * chunk, chunk)], osem
            )
        for j in range(n_chunks - nbuf, n_chunks):
            if j >= 0 and writes[j] is not None:
                writes[j].wait()

    return emb


def kernel(x, table):
    b, s = x.shape
    total = b * s
    d = table.shape[1]
    chunk = 32
    b_per_w = total // _NUM_WORKERS
    n_chunks = b_per_w // chunk
    x_grid = x.reshape(_NUM_WORKERS, n_chunks, chunk).astype(jnp.int32)
    out = _emb_lookup(total, d, chunk=chunk)(x_grid, table)
    return out.reshape(b, s, d)


# speedup vs baseline: 1.6805x; 1.0069x over previous
import functools

import jax
import jax.numpy as jnp
from jax import lax
from jax.experimental import pallas as pl
from jax.experimental.pallas import tpu as pltpu
from jax.experimental.pallas import tpu_sc as plsc

_NUM_CORES = 2
_NUM_SUBCORES = 16
_NUM_WORKERS = _NUM_CORES * _NUM_SUBCORES


def _emb_lookup(b, s, d, *, chunk, nbuf):
    total = b * s
    b_per_w = total // _NUM_WORKERS
    n_chunks = b_per_w // chunk
    w_per_row = s // b_per_w

    mesh = plsc.VectorSubcoreMesh(core_axis_name="c", subcore_axis_name="s")

    @functools.partial(
        pl.kernel,
        mesh=mesh,
        out_type=jax.ShapeDtypeStruct((b, s, d), jnp.float32),
        scratch_types=[
            pltpu.VMEM((b_per_w,), jnp.int32),
            [pltpu.VMEM((chunk, d), jnp.float32) for _ in range(nbuf)],
            pltpu.SemaphoreType.DMA,
            pltpu.SemaphoreType.DMA,
        ],
    )
    def emb(x_hbm, table_hbm, out_hbm, idx_v, bufs, gsem, osem):
        wid = lax.axis_index("s") * _NUM_CORES + lax.axis_index("c")
        row = wid // w_per_row
        col = (wid % w_per_row) * b_per_w
        pltpu.sync_copy(x_hbm.at[row, pl.ds(col, b_per_w)], idx_v)

        gathers = [None] * n_chunks
        writes = [None] * n_chunks
        for j in range(nbuf - 1):
            gathers[j] = pltpu.async_copy(
                table_hbm.at[idx_v.at[pl.ds(j * chunk, chunk)]],
                bufs[j % nbuf],
                gsem,
            )
        for j in range(n_chunks):
            gathers[j].wait()
            nxt = j + nbuf - 1
            if nxt < n_chunks:
                if writes[nxt - nbuf] is not None:
                    writes[nxt - nbuf].wait()
                gathers[nxt] = pltpu.async_copy(
                    table_hbm.at[idx_v.at[pl.ds(nxt * chunk, chunk)]],
                    bufs[nxt % nbuf],
                    gsem,
                )
            writes[j] = pltpu.async_copy(
                bufs[j % nbuf],
                out_hbm.at[row, pl.ds(col + j * chunk, chunk)],
                osem,
            )
        for j in range(n_chunks - nbuf, n_chunks):
            if j >= 0 and writes[j] is not None:
                writes[j].wait()

    return emb


def kernel(x, table):
    b, s = x.shape
    d = table.shape[1]
    return _emb_lookup(b, s, d, chunk=32, nbuf=4)(x, table)
